# baseline (speedup 1.0000x reference)
import jax
import jax.numpy as jnp
import numpy as np
from jax import lax
from jax.experimental import pallas as pl
from jax.experimental.pallas import tpu as pltpu
from jax.experimental.pallas import tpu_sc as plsc

B = 16
MAX_LEN = 2048
TOTAL = 16384
HALF = MAX_LEN // 2
NEG_INF = np.float32(-np.inf)


def _body(flatp_hbm, cu_hbm, out_hbm, cu_v, stage_v, buf_v, sem):
    s = lax.axis_index("s")

    pltpu.sync_copy(cu_hbm, cu_v)

    row_vec = jnp.full((16,), s, dtype=jnp.int32)
    cu_i = plsc.load_gather(cu_v, [row_vec])
    cu_i1 = plsc.load_gather(cu_v, [row_vec + 1])

    start = jnp.max(cu_i)
    s0 = pl.multiple_of(start & ~7, 8)
    pltpu.sync_copy(flatp_hbm.at[pl.ds(s0, MAX_LEN + 16)], stage_v)

    rem = cu_i1 - cu_i
    off = jnp.bitwise_and(cu_i, 7)
    lanes = lax.iota(jnp.int32, 16)
    for t in range(MAX_LEN // 16):
        li = off + (t * 16 + lanes)
        v = plsc.load_gather(stage_v, [li])
        valid = (t * 16 + lanes) < rem
        buf_v[pl.ds(t * 16, 16)] = jnp.where(valid & (v != 0.0), v, NEG_INF)

    pltpu.sync_copy(buf_v, out_hbm.at[s])


def kernel(flat, cu_seqlens):
    flatp = jnp.concatenate([flat, jnp.zeros((MAX_LEN,), flat.dtype)])
    mesh = plsc.VectorSubcoreMesh(
        core_axis_name="c", subcore_axis_name="s", num_cores=1, num_subcores=16
    )
    run = pl.kernel(
        _body,
        out_type=jax.ShapeDtypeStruct((B, MAX_LEN), jnp.float32),
        mesh=mesh,
        scratch_types=[
            pltpu.VMEM((B + 1,), jnp.int32),
            pltpu.VMEM((MAX_LEN + 16,), jnp.float32),
            pltpu.VMEM((MAX_LEN,), jnp.float32),
            pltpu.SemaphoreType.DMA,
        ],
        compiler_params=pltpu.CompilerParams(needs_layout_passes=False),
    )
    return run(flatp, cu_seqlens)


# speedup vs baseline: 1.0082x; 1.0082x over previous
import jax
import jax.numpy as jnp
import numpy as np
from jax import lax
from jax.experimental import pallas as pl
from jax.experimental.pallas import tpu as pltpu
from jax.experimental.pallas import tpu_sc as plsc

B = 16
MAX_LEN = 2048
TOTAL = 16384
HALF = MAX_LEN // 2
NEG_INF = np.float32(-np.inf)


def _body(flatp_hbm, cu_hbm, out_hbm, cu_v, stage_v, buf_v, sem):
    c = lax.axis_index("c")
    s = lax.axis_index("s")

    pltpu.sync_copy(cu_hbm, cu_v)

    row_vec = jnp.full((16,), s, dtype=jnp.int32)
    cu_i = plsc.load_gather(cu_v, [row_vec])
    cu_i1 = plsc.load_gather(cu_v, [row_vec + 1])

    c0 = c * HALF
    start = jnp.max(cu_i) + c0
    s0 = pl.multiple_of(start & ~7, 8)
    pltpu.sync_copy(flatp_hbm.at[pl.ds(s0, HALF + 16)], stage_v)

    rem = cu_i1 - (cu_i + c0)
    off = jnp.bitwise_and(cu_i + c0, 7)
    lanes = lax.iota(jnp.int32, 16)
    for t in range(HALF // 16):
        li = off + (t * 16 + lanes)
        v = plsc.load_gather(stage_v, [li])
        valid = (t * 16 + lanes) < rem
        buf_v[pl.ds(t * 16, 16)] = jnp.where(valid & (v != 0.0), v, NEG_INF)

    pltpu.sync_copy(buf_v, out_hbm.at[s, pl.ds(c0, HALF)])


def kernel(flat, cu_seqlens):
    flatp = jnp.concatenate([flat, jnp.zeros((MAX_LEN,), flat.dtype)])
    mesh = plsc.VectorSubcoreMesh(
        core_axis_name="c", subcore_axis_name="s", num_cores=2, num_subcores=16
    )
    run = pl.kernel(
        _body,
        out_type=jax.ShapeDtypeStruct((B, MAX_LEN), jnp.float32),
        mesh=mesh,
        scratch_types=[
            pltpu.VMEM((B + 1,), jnp.int32),
            pltpu.VMEM((HALF + 16,), jnp.float32),
            pltpu.VMEM((HALF,), jnp.float32),
            pltpu.SemaphoreType.DMA,
        ],
        compiler_params=pltpu.CompilerParams(needs_layout_passes=False),
    )
    return run(flatp, cu_seqlens)
